# baseline (speedup 1.0000x reference)
import functools

import jax
import jax.numpy as jnp
from jax import lax
from jax.experimental import pallas as pl
from jax.experimental.pallas import tpu as pltpu
from jax.experimental.pallas import tpu_sc as plsc

N_ROWS = 1000000
N_COLS = 64
N_IDX = 16384
NC, NS = 2, 16
COLS_PER_SC = N_COLS // NC
PER_T = N_IDX // NS
L = 16

ALIGNED = (N_ROWS // 128) * 128
TAIL = N_ROWS - ALIGNED
HALF = ALIGNED // 2

_mesh = plsc.VectorSubcoreMesh(core_axis_name="c", subcore_axis_name="s")


@functools.partial(
    pl.kernel,
    out_type=jax.ShapeDtypeStruct((N_COLS, N_IDX), jnp.float32),
    mesh=_mesh,
    scratch_types=[
        pltpu.VMEM_SHARED((N_ROWS,), jnp.float32),
        pltpu.VMEM_SHARED((N_ROWS,), jnp.float32),
        pltpu.VMEM_SHARED((N_COLS * 128,), jnp.float32),
        pltpu.VMEM((PER_T,), jnp.int32),
        pltpu.VMEM((PER_T,), jnp.int32),
        pltpu.VMEM((PER_T,), jnp.float32),
        pltpu.VMEM((PER_T,), jnp.float32),
        pltpu.SemaphoreType.DMA,
        pltpu.SemaphoreType.DMA,
    ],
)
def _gather(xt_hbm, idxt_hbm, tail_hbm, outt_hbm, row0, row1, tailspm, idxv,
            cidxv, datav, tdatav, ssem, gsem):
    cid = lax.axis_index("c")
    sid = lax.axis_index("s")
    j0 = cid * COLS_PER_SC

    def stage(j, buf):
        @pl.when(sid < 2)
        def _half():
            off = sid * HALF
            pltpu.async_copy(
                xt_hbm.at[j, pl.ds(off, HALF)],
                buf.at[pl.ds(off, HALF)],
                ssem,
            )

    def wait_stage(buf):
        @pl.when(sid < 2)
        def _half():
            off = sid * HALF
            pltpu.make_async_copy(
                xt_hbm.at[0, pl.ds(off, HALF)],
                buf.at[pl.ds(off, HALF)],
                ssem,
            ).wait()

    def serve(j, buf):
        pltpu.sync_copy(idxt_hbm.at[j, pl.ds(sid * PER_T, PER_T)], idxv)

        def mkpatch(t, carry):
            sl = pl.ds(t * L, L)
            rv = idxv[sl]
            m = rv >= ALIGNED
            cidxv[sl] = jnp.where(m, rv - (ALIGNED - j * 128), 0)
            return carry

        lax.fori_loop(0, PER_T // L, mkpatch, 0)

        pltpu.async_copy(buf.at[idxv], datav, gsem)
        pltpu.async_copy(tailspm.at[cidxv], tdatav, gsem)
        pltpu.make_async_copy(buf.at[idxv], datav, gsem).wait()
        pltpu.make_async_copy(tailspm.at[cidxv], tdatav, gsem).wait()

        def merge(t, carry):
            sl = pl.ds(t * L, L)
            m = idxv[sl] >= ALIGNED
            datav[sl] = jnp.where(m, tdatav[sl], datav[sl])
            return carry

        lax.fori_loop(0, PER_T // L, merge, 0)
        pltpu.sync_copy(datav, outt_hbm.at[j, pl.ds(sid * PER_T, PER_T)])

    @pl.when(sid == 0)
    def _load_tail():
        pltpu.sync_copy(tail_hbm, tailspm)

    stage(j0, row0)

    def pair_body(i, carry):
        j = j0 + 2 * i

        wait_stage(row0)
        plsc.subcore_barrier()
        stage(j + 1, row1)
        serve(j, row0)
        plsc.subcore_barrier()

        wait_stage(row1)
        plsc.subcore_barrier()

        @pl.when(i < COLS_PER_SC // 2 - 1)
        def _s0():
            stage(j + 2, row0)

        serve(j + 1, row1)
        plsc.subcore_barrier()
        return carry

    lax.fori_loop(0, COLS_PER_SC // 2, pair_body, 0)


def kernel(x, idx):
    tail = jnp.pad(x[ALIGNED:].T, ((0, 0), (0, 128 - TAIL))).reshape(-1)
    return _gather(x.T, idx.T, tail).T


# speedup vs baseline: 1.9008x; 1.9008x over previous
import functools

import jax
import jax.numpy as jnp
from jax import lax
from jax.experimental import pallas as pl
from jax.experimental.pallas import tpu as pltpu
from jax.experimental.pallas import tpu_sc as plsc

N_ROWS = 1000000
N_COLS = 64
N_IDX = 16384
NC, NS = 2, 16
COLS_PER_SC = N_COLS // NC
PER_T = N_IDX // NS

_mesh = plsc.VectorSubcoreMesh(core_axis_name="c", subcore_axis_name="s")


@functools.partial(
    pl.kernel,
    out_type=jax.ShapeDtypeStruct((N_COLS, N_IDX), jnp.float32),
    mesh=_mesh,
    scratch_types=[
        pltpu.VMEM_SHARED((N_ROWS,), jnp.float32),
        pltpu.VMEM_SHARED((N_ROWS,), jnp.float32),
        pltpu.VMEM((PER_T,), jnp.int32),
        pltpu.VMEM((PER_T,), jnp.float32),
        pltpu.SemaphoreType.DMA,
        pltpu.SemaphoreType.DMA,
    ],
)
def _gather(xt_hbm, idxt_hbm, outt_hbm, row0, row1, idxv, datav, ssem, gsem):
    cid = lax.axis_index("c")
    sid = lax.axis_index("s")
    j0 = cid * COLS_PER_SC

    def stage(j, buf):
        pltpu.async_copy(xt_hbm.at[j], buf, ssem)

    def serve(j, buf):
        pltpu.sync_copy(idxt_hbm.at[j, pl.ds(sid * PER_T, PER_T)], idxv)
        pltpu.async_copy(buf.at[idxv], datav, gsem).wait()
        pltpu.sync_copy(datav, outt_hbm.at[j, pl.ds(sid * PER_T, PER_T)])

    def wait_stage(buf):
        pltpu.make_async_copy(xt_hbm.at[0], buf, ssem).wait()

    @pl.when(sid == 0)
    def _prologue():
        stage(j0, row0)

    def pair_body(i, carry):
        j = j0 + 2 * i

        @pl.when(sid == 0)
        def _w0():
            wait_stage(row0)

        plsc.subcore_barrier()

        @pl.when(sid == 0)
        def _s1():
            stage(j + 1, row1)

        serve(j, row0)
        plsc.subcore_barrier()

        @pl.when(sid == 0)
        def _w1():
            wait_stage(row1)

        plsc.subcore_barrier()

        @pl.when(jnp.logical_and(sid == 0, i < COLS_PER_SC // 2 - 1))
        def _s0():
            stage(j + 2, row0)

        serve(j + 1, row1)
        plsc.subcore_barrier()
        return carry

    lax.fori_loop(0, COLS_PER_SC // 2, pair_body, 0)


def kernel(x, idx):
    return _gather(x.T, idx.T).T
